# baseline (speedup 1.0000x reference)
import functools

import jax
import jax.numpy as jnp
from jax import lax
from jax.experimental import pallas as pl
from jax.experimental.pallas import tpu as pltpu
from jax.experimental.pallas import tpu_sc as plsc

N_NODES = 10000
IN_CH = 128
HID = 16
OUT_CH = 128
E = 320000

NC, NS, LANES = 2, 16, 16
NW = NC * NS
E_W = E // NW
BLK = 125
NBLK = 80
ROWS_PER_SUB = N_NODES // NS


ROWS_STAGE = N_NODES // NS


def _seg_sum_body(y_hbm, src_hbm, dst_hbm, out_hbm,
                  src_v, dst_v, bufa, bufb, zrow_v, y_sh, acc_sh,
                  sema, semb):
    c = lax.axis_index("c")
    s = lax.axis_index("s")
    wid = s * NC + c

    pltpu.sync_copy(src_hbm.at[wid], src_v)
    pltpu.sync_copy(dst_hbm.at[wid], dst_v)
    pltpu.sync_copy(y_hbm.at[pl.ds(s * ROWS_STAGE, ROWS_STAGE)],
                    y_sh.at[pl.ds(s * ROWS_STAGE, ROWS_STAGE)])

    def zbody(i, carry):
        zrow_v[i, :] = jnp.zeros((LANES,), jnp.float32)
        return carry
    lax.fori_loop(0, ROWS_PER_SUB, zbody, 0)
    pltpu.sync_copy(zrow_v, acc_sh.at[pl.ds(s * ROWS_PER_SUB, ROWS_PER_SUB)])
    plsc.subcore_barrier()

    pltpu.async_copy(y_sh.at[src_v.at[0]], bufa, sema)

    def ebody(jj, carry):
        j0 = 2 * jj
        pltpu.async_copy(y_sh.at[src_v.at[j0 + 1]], bufb, semb)
        pltpu.make_async_copy(y_sh.at[src_v.at[j0]], bufa, sema).wait()
        pltpu.sync_copy(bufa, acc_sh.at[dst_v.at[j0]], add=True)

        @pl.when(jj + 1 < NBLK // 2)
        def _():
            pltpu.async_copy(y_sh.at[src_v.at[j0 + 2]], bufa, sema)

        pltpu.make_async_copy(y_sh.at[src_v.at[j0 + 1]], bufb, semb).wait()
        pltpu.sync_copy(bufb, acc_sh.at[dst_v.at[j0 + 1]], add=True)
        return carry
    lax.fori_loop(0, NBLK // 2, ebody, 0)
    plsc.subcore_barrier()

    pltpu.sync_copy(acc_sh.at[pl.ds(s * ROWS_PER_SUB, ROWS_PER_SUB)],
                    out_hbm.at[c, pl.ds(s * ROWS_PER_SUB, ROWS_PER_SUB)])


def _seg_sum(y, srcw, dstw):
    mesh = plsc.VectorSubcoreMesh(core_axis_name="c", subcore_axis_name="s",
                                  num_cores=NC, num_subcores=NS)
    return pl.kernel(
        _seg_sum_body,
        out_type=jax.ShapeDtypeStruct((NC, N_NODES, LANES), jnp.float32),
        mesh=mesh,
        scratch_types=[
            pltpu.VMEM((NBLK, BLK), jnp.int32),
            pltpu.VMEM((NBLK, BLK), jnp.int32),
            pltpu.VMEM((BLK, LANES), jnp.float32),
            pltpu.VMEM((BLK, LANES), jnp.float32),
            pltpu.VMEM((ROWS_PER_SUB, LANES), jnp.float32),
            pltpu.VMEM_SHARED((N_NODES, LANES), jnp.float32),
            pltpu.VMEM_SHARED((N_NODES, LANES), jnp.float32),
            pltpu.SemaphoreType.DMA,
            pltpu.SemaphoreType.DMA,
        ],
        compiler_params=pltpu.CompilerParams(use_tc_tiling_on_sc=False),
    )(y, srcw, dstw)


def _mm1(x, W1a):
    def body(x_ref, w_ref, o_ref):
        o_ref[...] = jnp.dot(x_ref[...], w_ref[...],
                             preferred_element_type=jnp.float32)
    return pl.pallas_call(
        body,
        out_shape=jax.ShapeDtypeStruct((N_NODES, HID), jnp.float32),
    )(x, W1a)


def _mid(y, parts, b1a, W2a, b2a):
    def body(y_ref, p_ref, b1_ref, w2_ref, b2_ref, o_ref):
        agg = p_ref[0, :N_NODES, :] + p_ref[1, :N_NODES, :]
        u = jnp.maximum(y_ref[...] + agg + b1_ref[...], 0.0)
        v = jnp.dot(u, w2_ref[...], preferred_element_type=jnp.float32)
        o_ref[...] = jnp.maximum(v + b2_ref[...], 0.0)
    return pl.pallas_call(
        body,
        out_shape=jax.ShapeDtypeStruct((N_NODES, HID), jnp.float32),
    )(y, parts, b1a, W2a, b2a)


def _final(h1, parts, W1b, b1b, W2b, b2b):
    def body(h_ref, p_ref, w1_ref, b1_ref, w2_ref, b2_ref, o_ref):
        agg = p_ref[0, :N_NODES, :] + p_ref[1, :N_NODES, :]
        g = h_ref[...] + agg
        t = jnp.dot(g, w1_ref[...], preferred_element_type=jnp.float32)
        t = jnp.maximum(t + b1_ref[...], 0.0)
        o_ref[...] = jnp.dot(t, w2_ref[...],
                             preferred_element_type=jnp.float32) + b2_ref[...]
    return pl.pallas_call(
        body,
        out_shape=jax.ShapeDtypeStruct((N_NODES, OUT_CH), jnp.float32),
    )(h1, parts, W1b, b1b, W2b, b2b)


def kernel(x, edge_index, W1a, b1a, W2a, b2a, W1b, b1b, W2b, b2b):
    eiw = edge_index.astype(jnp.int32).reshape(2, NW, NBLK, BLK)
    srcw = eiw[0]
    dstw = eiw[1]

    y = _mm1(x, W1a)
    p1 = _seg_sum(y, srcw, dstw)
    h1 = _mid(y, p1, b1a.reshape(1, HID), W2a, b2a.reshape(1, HID))
    p2 = _seg_sum(h1, srcw, dstw)
    return _final(h1, p2, W1b, b1b.reshape(1, OUT_CH), W2b,
                  b2b.reshape(1, OUT_CH))


# speedup vs baseline: 1.5339x; 1.5339x over previous
import jax
import jax.numpy as jnp
from jax import lax
from jax.experimental import pallas as pl
from jax.experimental.pallas import tpu as pltpu
from jax.experimental.pallas import tpu_sc as plsc

N_NODES = 10000
IN_CH = 128
HID = 16
OUT_CH = 128
E = 320000

NC, NS, LANES = 2, 16, 16
NW = NC * NS
BLK = 128
NBLK_TOT = E // BLK
NBLK_BASE = NBLK_TOT // NW
NBLK_XTRA = NBLK_TOT % NW
NBLK_MAX = NBLK_BASE + 1
PK = N_NODES // 8
ROWS_PER_SUB = N_NODES // NS


def _seg_sum_body(y_hbm, ei_hbm, out_hbm,
                  src_v, dst_v, bufa, bufb, zrow_v, y_sh, acc_sh,
                  sema, semb):
    c = lax.axis_index("c")
    s = lax.axis_index("s")
    wid = s * NC + c
    nblk = NBLK_BASE + jnp.where(wid < NBLK_XTRA, 1, 0)

    base = NBLK_BASE * wid
    pltpu.sync_copy(ei_hbm.at[pl.ds(base, NBLK_BASE)],
                    src_v.at[pl.ds(0, NBLK_BASE)])
    pltpu.sync_copy(ei_hbm.at[pl.ds(NBLK_TOT + base, NBLK_BASE)],
                    dst_v.at[pl.ds(0, NBLK_BASE)])

    @pl.when(wid < NBLK_XTRA)
    def _():
        xrow = NBLK_BASE * NW + wid
        pltpu.sync_copy(ei_hbm.at[xrow], src_v.at[NBLK_BASE])
        pltpu.sync_copy(ei_hbm.at[NBLK_TOT + xrow], dst_v.at[NBLK_BASE])

    pltpu.sync_copy(y_hbm.at[pl.ds(s * ROWS_PER_SUB, ROWS_PER_SUB)],
                    y_sh.at[pl.ds(s * ROWS_PER_SUB, ROWS_PER_SUB)])

    def zbody(i, carry):
        zrow_v[i, :] = jnp.zeros((LANES,), jnp.float32)
        return carry
    lax.fori_loop(0, ROWS_PER_SUB, zbody, 0)
    pltpu.sync_copy(zrow_v, acc_sh.at[pl.ds(s * ROWS_PER_SUB, ROWS_PER_SUB)])
    plsc.subcore_barrier()

    pltpu.async_copy(y_sh.at[src_v.at[0]], bufa, sema)
    pltpu.async_copy(y_sh.at[src_v.at[1]], bufb, semb)

    def ebody(j, carry):
        @pl.when((j & 1) == 0)
        def _():
            pltpu.make_async_copy(y_sh.at[src_v.at[j]], bufa, sema).wait()
            pltpu.sync_copy(bufa, acc_sh.at[dst_v.at[j]], add=True)

            @pl.when(j + 2 < nblk)
            def _():
                pltpu.async_copy(y_sh.at[src_v.at[j + 2]], bufa, sema)

        @pl.when((j & 1) == 1)
        def _():
            pltpu.make_async_copy(y_sh.at[src_v.at[j]], bufb, semb).wait()
            pltpu.sync_copy(bufb, acc_sh.at[dst_v.at[j]], add=True)

            @pl.when(j + 2 < nblk)
            def _():
                pltpu.async_copy(y_sh.at[src_v.at[j + 2]], bufb, semb)

        return carry
    lax.fori_loop(0, nblk, ebody, 0)
    plsc.subcore_barrier()

    pltpu.sync_copy(acc_sh.at[pl.ds(s * ROWS_PER_SUB, ROWS_PER_SUB)],
                    out_hbm.at[c, pl.ds(s * ROWS_PER_SUB, ROWS_PER_SUB)])


def _seg_sum(y_lin, ei_v):
    mesh = plsc.VectorSubcoreMesh(core_axis_name="c", subcore_axis_name="s",
                                  num_cores=NC, num_subcores=NS)
    return pl.kernel(
        _seg_sum_body,
        out_type=jax.ShapeDtypeStruct((NC, N_NODES, LANES), jnp.float32),
        mesh=mesh,
        scratch_types=[
            pltpu.VMEM((NBLK_MAX, BLK), jnp.int32),
            pltpu.VMEM((NBLK_MAX, BLK), jnp.int32),
            pltpu.VMEM((BLK, LANES), jnp.float32),
            pltpu.VMEM((BLK, LANES), jnp.float32),
            pltpu.VMEM((ROWS_PER_SUB, LANES), jnp.float32),
            pltpu.VMEM_SHARED((N_NODES, LANES), jnp.float32),
            pltpu.VMEM_SHARED((N_NODES, LANES), jnp.float32),
            pltpu.SemaphoreType.DMA,
            pltpu.SemaphoreType.DMA,
        ],
        compiler_params=pltpu.CompilerParams(use_tc_tiling_on_sc=False),
    )(y_lin, ei_v)


def _block_eq_mask(rows, cols, rblk, cblk):
    ii = lax.broadcasted_iota(jnp.int32, (rows, cols), 0) // rblk
    jj = lax.broadcasted_iota(jnp.int32, (rows, cols), 1) // cblk
    return ii == jj


def _mm1(x, W1a):
    def body(x_ref, w_ref, o_ref):
        x2 = x_ref[...].reshape(PK, 8 * IN_CH)
        wstack = jnp.where(_block_eq_mask(8 * IN_CH, 128, IN_CH, HID),
                           jnp.tile(w_ref[...], (8, 8)), 0.0)
        o_ref[...] = jnp.dot(x2, wstack, preferred_element_type=jnp.float32)
    return pl.pallas_call(
        body,
        out_shape=jax.ShapeDtypeStruct((PK, 128), jnp.float32),
    )(x, W1a)


def _mid(y_pk, parts_pk, b1a, W2a, b2a):
    def body(y_ref, p_ref, b1_ref, w2_ref, b2_ref, o_ref):
        u = y_ref[...] + p_ref[0] + p_ref[1] + jnp.tile(b1_ref[...], (1, 8))
        u = jnp.maximum(u, 0.0)
        w2blk = jnp.where(_block_eq_mask(128, 128, HID, HID),
                          jnp.tile(w2_ref[...], (8, 8)), 0.0)
        v = jnp.dot(u, w2blk, preferred_element_type=jnp.float32)
        o_ref[...] = jnp.maximum(v + jnp.tile(b2_ref[...], (1, 8)), 0.0)
    return pl.pallas_call(
        body,
        out_shape=jax.ShapeDtypeStruct((PK, 128), jnp.float32),
    )(y_pk, parts_pk, b1a, W2a, b2a)


def _final(h_pk, parts_pk, W1b, b1b, W2b, b2b):
    def body(h_ref, p_ref, w1_ref, b1_ref, w2_ref, b2_ref, o_ref):
        g = h_ref[...] + p_ref[0] + p_ref[1]
        w1exp = jnp.where(_block_eq_mask(128, 8 * OUT_CH, HID, OUT_CH),
                          jnp.tile(w1_ref[...], (8, 8)), 0.0)
        t = jnp.dot(g, w1exp, preferred_element_type=jnp.float32)
        t = t.reshape(N_NODES, OUT_CH) + b1_ref[...]
        t = jnp.maximum(t, 0.0)
        o_ref[...] = jnp.dot(t, w2_ref[...],
                             preferred_element_type=jnp.float32) + b2_ref[...]
    return pl.pallas_call(
        body,
        out_shape=jax.ShapeDtypeStruct((N_NODES, OUT_CH), jnp.float32),
    )(h_pk, parts_pk, W1b, b1b, W2b, b2b)


def kernel(x, edge_index, W1a, b1a, W2a, b2a, W1b, b1b, W2b, b2b):
    ei_v = edge_index.astype(jnp.int32).reshape(2 * NBLK_TOT, BLK)

    y_pk = _mm1(x, W1a)
    p1 = _seg_sum(y_pk.reshape(N_NODES, HID), ei_v)
    h1_pk = _mid(y_pk, p1.reshape(NC, PK, 128), b1a.reshape(1, HID), W2a,
                 b2a.reshape(1, HID))
    p2 = _seg_sum(h1_pk.reshape(N_NODES, HID), ei_v)
    return _final(h1_pk, p2.reshape(NC, PK, 128), W1b, b1b.reshape(1, OUT_CH),
                  W2b, b2b.reshape(1, OUT_CH))


# speedup vs baseline: 1.7718x; 1.1551x over previous
import jax
import jax.numpy as jnp
from jax import lax
from jax.experimental import pallas as pl
from jax.experimental.pallas import tpu as pltpu
from jax.experimental.pallas import tpu_sc as plsc

N_NODES = 10000
IN_CH = 128
HID = 16
OUT_CH = 128
E = 320000

NC, NS, LANES = 2, 16, 16
NW = NC * NS
BLK = 128
NBLK_TOT = E // BLK
NBLK_BASE = NBLK_TOT // NW
NBLK_XTRA = NBLK_TOT % NW
NBLK_MAX = NBLK_BASE + 1
PK = N_NODES // 8
ROWS_PER_SUB = N_NODES // NS


NCHUNK = 4
CH_E = NBLK_BASE * BLK // NCHUNK
ZROWS = 125


def _seg_sum_body(y_hbm, ei_hbm, out_hbm,
                  src_v, dst_v, tidx_v, bufa, bufb, tailbuf, zrow_v,
                  y_sh, acc_sh, sema, semb, semst, semt):
    c = lax.axis_index("c")
    s = lax.axis_index("s")
    wid = s * NC + c
    base = NBLK_BASE * BLK * wid

    sts = []
    for k in range(NCHUNK):
        sts.append(pltpu.async_copy(
            ei_hbm.at[pl.ds(base + k * CH_E, CH_E)], src_v.at[k], semst))
        sts.append(pltpu.async_copy(
            ei_hbm.at[pl.ds(E + base + k * CH_E, CH_E)], dst_v.at[k], semst))
    sts.append(pltpu.async_copy(
        y_hbm.at[pl.ds(s * ROWS_PER_SUB, ROWS_PER_SUB)],
        y_sh.at[pl.ds(s * ROWS_PER_SUB, ROWS_PER_SUB)], semst))

    @plsc.parallel_loop(0, ZROWS, step=1, unroll=8)
    def _(i):
        zrow_v[i, :] = jnp.zeros((LANES,), jnp.float32)

    for st in sts:
        st.wait()
    for t in range(ROWS_PER_SUB // ZROWS):
        pltpu.sync_copy(zrow_v,
                        acc_sh.at[pl.ds(s * ROWS_PER_SUB + t * ZROWS, ZROWS)])
    plsc.subcore_barrier()

    @pl.when(wid < NBLK_XTRA)
    def _():
        xoff = NBLK_BASE * BLK * NW + wid * BLK
        pltpu.sync_copy(ei_hbm.at[pl.ds(xoff, BLK)], tidx_v.at[0])
        pltpu.sync_copy(ei_hbm.at[pl.ds(E + xoff, BLK)], tidx_v.at[1])
        pltpu.async_copy(y_sh.at[tidx_v.at[0]], tailbuf, semt).wait()
        pltpu.sync_copy(tailbuf, acc_sh.at[tidx_v.at[1]], add=True)

    pltpu.async_copy(y_sh.at[src_v.at[0]], bufa, sema)
    pltpu.async_copy(y_sh.at[src_v.at[1]], bufb, semb)
    for k in range(NCHUNK):
        buf, sem = (bufa, sema) if k % 2 == 0 else (bufb, semb)
        pltpu.make_async_copy(y_sh.at[src_v.at[k]], buf, sem).wait()
        pltpu.sync_copy(buf, acc_sh.at[dst_v.at[k]], add=True)
        if k + 2 < NCHUNK:
            pltpu.async_copy(y_sh.at[src_v.at[k + 2]], buf, sem)
    plsc.subcore_barrier()

    pltpu.sync_copy(acc_sh.at[pl.ds(s * ROWS_PER_SUB, ROWS_PER_SUB)],
                    out_hbm.at[c, pl.ds(s * ROWS_PER_SUB, ROWS_PER_SUB)])


def _seg_sum(y_lin, ei_v):
    mesh = plsc.VectorSubcoreMesh(core_axis_name="c", subcore_axis_name="s",
                                  num_cores=NC, num_subcores=NS)
    return pl.kernel(
        _seg_sum_body,
        out_type=jax.ShapeDtypeStruct((NC, N_NODES, LANES), jnp.float32),
        mesh=mesh,
        scratch_types=[
            pltpu.VMEM((NCHUNK, CH_E), jnp.int32),
            pltpu.VMEM((NCHUNK, CH_E), jnp.int32),
            pltpu.VMEM((2, BLK), jnp.int32),
            pltpu.VMEM((CH_E, LANES), jnp.float32),
            pltpu.VMEM((CH_E, LANES), jnp.float32),
            pltpu.VMEM((BLK, LANES), jnp.float32),
            pltpu.VMEM((ZROWS, LANES), jnp.float32),
            pltpu.VMEM_SHARED((N_NODES, LANES), jnp.float32),
            pltpu.VMEM_SHARED((N_NODES, LANES), jnp.float32),
            pltpu.SemaphoreType.DMA,
            pltpu.SemaphoreType.DMA,
            pltpu.SemaphoreType.DMA,
            pltpu.SemaphoreType.DMA,
        ],
        compiler_params=pltpu.CompilerParams(use_tc_tiling_on_sc=False),
    )(y_lin, ei_v)


def _block_eq_mask(rows, cols, rblk, cblk):
    ii = lax.broadcasted_iota(jnp.int32, (rows, cols), 0) // rblk
    jj = lax.broadcasted_iota(jnp.int32, (rows, cols), 1) // cblk
    return ii == jj


def _mm1(x, W1a):
    def body(x_ref, w_ref, o_ref):
        x2 = x_ref[...].reshape(PK, 8 * IN_CH)
        wstack = jnp.where(_block_eq_mask(8 * IN_CH, 128, IN_CH, HID),
                           jnp.tile(w_ref[...], (8, 8)), 0.0)
        o_ref[...] = jnp.dot(x2, wstack, preferred_element_type=jnp.float32)
    return pl.pallas_call(
        body,
        out_shape=jax.ShapeDtypeStruct((PK, 128), jnp.float32),
    )(x, W1a)


def _mid(y_pk, parts_pk, b1a, W2a, b2a):
    def body(y_ref, p_ref, b1_ref, w2_ref, b2_ref, o_ref):
        u = y_ref[...] + p_ref[0] + p_ref[1] + jnp.tile(b1_ref[...], (1, 8))
        u = jnp.maximum(u, 0.0)
        w2blk = jnp.where(_block_eq_mask(128, 128, HID, HID),
                          jnp.tile(w2_ref[...], (8, 8)), 0.0)
        v = jnp.dot(u, w2blk, preferred_element_type=jnp.float32)
        o_ref[...] = jnp.maximum(v + jnp.tile(b2_ref[...], (1, 8)), 0.0)
    return pl.pallas_call(
        body,
        out_shape=jax.ShapeDtypeStruct((PK, 128), jnp.float32),
    )(y_pk, parts_pk, b1a, W2a, b2a)


def _final(h_pk, parts_pk, W1b, b1b, W2b, b2b):
    def body(h_ref, p_ref, w1_ref, b1_ref, w2_ref, b2_ref, o_ref):
        g = h_ref[...] + p_ref[0] + p_ref[1]
        w1exp = jnp.where(_block_eq_mask(128, 8 * OUT_CH, HID, OUT_CH),
                          jnp.tile(w1_ref[...], (8, 8)), 0.0)
        t = jnp.dot(g, w1exp, preferred_element_type=jnp.float32)
        t = t.reshape(N_NODES, OUT_CH) + b1_ref[...]
        t = jnp.maximum(t, 0.0)
        o_ref[...] = jnp.dot(t, w2_ref[...],
                             preferred_element_type=jnp.float32) + b2_ref[...]
    return pl.pallas_call(
        body,
        out_shape=jax.ShapeDtypeStruct((N_NODES, OUT_CH), jnp.float32),
    )(h_pk, parts_pk, W1b, b1b, W2b, b2b)


def kernel(x, edge_index, W1a, b1a, W2a, b2a, W1b, b1b, W2b, b2b):
    ei_v = edge_index.astype(jnp.int32).reshape(2 * E)

    y_pk = _mm1(x, W1a)
    p1 = _seg_sum(y_pk.reshape(N_NODES, HID), ei_v)
    h1_pk = _mid(y_pk, p1.reshape(NC, PK, 128), b1a.reshape(1, HID), W2a,
                 b2a.reshape(1, HID))
    p2 = _seg_sum(h1_pk.reshape(N_NODES, HID), ei_v)
    return _final(h1_pk, p2.reshape(NC, PK, 128), W1b, b1b.reshape(1, OUT_CH),
                  W2b, b2b.reshape(1, OUT_CH))
